# Initial kernel scaffold; baseline (speedup 1.0000x reference)
import jax
import jax.numpy as jnp
from jax.experimental import pallas as pl


def kernel(embeddings, labels):
    raise NotImplementedError("write your pallas kernel here")


# speedup vs baseline: 2.5523x; 2.5523x over previous
import functools

import jax
import jax.numpy as jnp
from jax.experimental import pallas as pl
from jax.experimental.pallas import tpu as pltpu

_LANE = 128
_VMEM_LIMIT = 56 * 1024 * 1024


def _round_up(x, m):
    return ((x + m - 1) // m) * m


def _dist_kernel(e_ref, sqc_ref, sqr_ref, o_ref, *, tm, tn):
    i = pl.program_id(0)
    j = pl.program_id(1)
    ei = e_ref[pl.ds(i * tm, tm), :]
    ej = e_ref[pl.ds(j * tn, tn), :]
    gram = jax.lax.dot_general(
        ei,
        ej,
        dimension_numbers=(((1,), (1,)), ((), ())),
        preferred_element_type=jnp.float32,
    )
    o_ref[...] = sqc_ref[...] + sqr_ref[...] - 2.0 * gram


def _choose_tiles(n_pad):
    if n_pad % 1024 == 0 and n_pad >= 2048:
        return 1024, 1024
    if n_pad % 512 == 0 and n_pad >= 1024:
        return 512, 512
    return n_pad, n_pad


def kernel(embeddings, labels):
    n, d = embeddings.shape
    d_pad = _round_up(d, _LANE)
    n_pad = _round_up(n, 1024) if n > 512 else _round_up(n, _LANE)
    tm, tn = _choose_tiles(n_pad)

    e_pad = jnp.zeros((n_pad, d_pad), jnp.float32).at[:n, :d].set(
        embeddings.astype(jnp.float32))
    sq = jnp.sum(e_pad * e_pad, axis=1)
    sq_col = sq.reshape(n_pad, 1)
    sq_row = sq.reshape(1, n_pad)
    e_bf = e_pad.astype(jnp.bfloat16)

    dist = pl.pallas_call(
        functools.partial(_dist_kernel, tm=tm, tn=tn),
        out_shape=jax.ShapeDtypeStruct((n_pad, n_pad), jnp.float32),
        grid=(n_pad // tm, n_pad // tn),
        in_specs=[
            pl.BlockSpec((n_pad, d_pad), lambda i, j: (0, 0)),
            pl.BlockSpec((tm, 1), lambda i, j: (i, 0)),
            pl.BlockSpec((1, tn), lambda i, j: (0, j)),
        ],
        out_specs=pl.BlockSpec((tm, tn), lambda i, j: (i, j)),
        compiler_params=pltpu.CompilerParams(
            dimension_semantics=("parallel", "parallel"),
            vmem_limit_bytes=_VMEM_LIMIT,
        ),
    )(e_bf, sq_col, sq_row)
    return dist[:n, :n]


# speedup vs baseline: 2.5734x; 1.0083x over previous
import functools

import jax
import jax.numpy as jnp
from jax.experimental import pallas as pl
from jax.experimental.pallas import tpu as pltpu

_LANE = 128
_VMEM_LIMIT = 56 * 1024 * 1024


def _round_up(x, m):
    return ((x + m - 1) // m) * m


def _dist_kernel(e_ref, sqc_ref, sqr_ref, o_ref, *, tm, tn):
    i = pl.program_id(0)
    j = pl.program_id(1)
    ei = e_ref[pl.ds(i * tm, tm), :].astype(jnp.bfloat16)
    ej = e_ref[pl.ds(j * tn, tn), :].astype(jnp.bfloat16)
    gram = jax.lax.dot_general(
        ei,
        ej,
        dimension_numbers=(((1,), (1,)), ((), ())),
        preferred_element_type=jnp.float32,
    )
    o_ref[...] = sqc_ref[...] + sqr_ref[...] - 2.0 * gram


def _choose_tiles(n_pad):
    if n_pad % 1024 == 0 and n_pad >= 2048:
        return 1024, 1024
    if n_pad % 512 == 0 and n_pad >= 1024:
        return 512, 512
    return n_pad, n_pad


def kernel(embeddings, labels):
    n, d = embeddings.shape
    d_pad = _round_up(d, _LANE)
    n_pad = _round_up(n, 1024) if n > 512 else _round_up(n, _LANE)
    tm, tn = _choose_tiles(n_pad)

    e32 = embeddings.astype(jnp.float32)
    if (n_pad, d_pad) == (n, d):
        e_pad = e32
    else:
        e_pad = jnp.zeros((n_pad, d_pad), jnp.float32).at[:n, :d].set(e32)
    sq = jnp.sum(e_pad * e_pad, axis=1)
    sq_col = sq.reshape(n_pad, 1)
    sq_row = sq.reshape(1, n_pad)

    dist = pl.pallas_call(
        functools.partial(_dist_kernel, tm=tm, tn=tn),
        out_shape=jax.ShapeDtypeStruct((n_pad, n_pad), jnp.float32),
        grid=(n_pad // tm, n_pad // tn),
        in_specs=[
            pl.BlockSpec((n_pad, d_pad), lambda i, j: (0, 0)),
            pl.BlockSpec((tm, 1), lambda i, j: (i, 0)),
            pl.BlockSpec((1, tn), lambda i, j: (0, j)),
        ],
        out_specs=pl.BlockSpec((tm, tn), lambda i, j: (i, j)),
        compiler_params=pltpu.CompilerParams(
            dimension_semantics=("parallel", "parallel"),
            vmem_limit_bytes=_VMEM_LIMIT,
        ),
    )(e_pad, sq_col, sq_row)
    return dist[:n, :n]


# speedup vs baseline: 2.9086x; 1.1302x over previous
import functools

import jax
import jax.numpy as jnp
from jax.experimental import pallas as pl
from jax.experimental.pallas import tpu as pltpu

_LANE = 128
_SUBLANE_MIN = 8
_VMEM_LIMIT = 60 * 1024 * 1024


def _round_up(x, m):
    return ((x + m - 1) // m) * m


def _dist_kernel(e_ref, o_ref, ebf_ref, sqc_ref, sqr_ref, *, tm, per_core):
    c = pl.program_id(0)
    s = pl.program_id(1)

    @pl.when(s == 0)
    def _init():
        e32 = e_ref[...]
        ebf_ref[...] = e32.astype(jnp.bfloat16)
        sq = jnp.sum(e32 * e32, axis=1, keepdims=True)
        sqc_ref[...] = sq
        sqr_ref[...] = jnp.transpose(sq, (1, 0))

    i = c * per_core + s
    ei = ebf_ref[pl.ds(i * tm, tm), :]
    gram = jax.lax.dot_general(
        ei,
        ebf_ref[...],
        dimension_numbers=(((1,), (1,)), ((), ())),
        preferred_element_type=jnp.float32,
    )
    o_ref[...] = sqc_ref[pl.ds(i * tm, tm), :] + sqr_ref[...] - 2.0 * gram


def kernel(embeddings, labels):
    n, d = embeddings.shape
    d_pad = _round_up(d, _LANE)
    tm = 512 if n > 512 else _round_up(n, _SUBLANE_MIN)
    n_pad = _round_up(n, 2 * tm)
    n_blocks = n_pad // tm
    per_core = n_blocks // 2

    e32 = embeddings.astype(jnp.float32)
    if (n_pad, d_pad) == (n, d):
        e_pad = e32
    else:
        e_pad = jnp.zeros((n_pad, d_pad), jnp.float32).at[:n, :d].set(e32)

    dist = pl.pallas_call(
        functools.partial(_dist_kernel, tm=tm, per_core=per_core),
        out_shape=jax.ShapeDtypeStruct((n_pad, n_pad), jnp.float32),
        grid=(2, per_core),
        in_specs=[
            pl.BlockSpec((n_pad, d_pad), lambda c, s: (0, 0)),
        ],
        out_specs=pl.BlockSpec((tm, n_pad), lambda c, s, pc=per_core: (c * pc + s, 0)),
        scratch_shapes=[
            pltpu.VMEM((n_pad, d_pad), jnp.bfloat16),
            pltpu.VMEM((n_pad, 1), jnp.float32),
            pltpu.VMEM((1, n_pad), jnp.float32),
        ],
        compiler_params=pltpu.CompilerParams(
            dimension_semantics=("parallel", "arbitrary"),
            vmem_limit_bytes=_VMEM_LIMIT,
        ),
    )(e_pad)
    return dist[:n, :n]
